# baseline (speedup 1.0000x reference)
import functools

import jax
import jax.numpy as jnp
from jax import lax
from jax.experimental import pallas as pl
from jax.experimental.pallas import tpu as pltpu
from jax.experimental.pallas import tpu_sc as plsc

NC = 2
NS = 16
NW = NC * NS

EMB_DIM = 64
BLK = 128
DT = EMB_DIM // 8


def _make_gather(seq: int, btiles: int):
    nblk = seq * btiles
    assert nblk % (2 * NW) == 0
    r = nblk // NW

    mesh = plsc.VectorSubcoreMesh(core_axis_name="c", subcore_axis_name="s")

    @functools.partial(
        pl.kernel,
        out_type=jax.ShapeDtypeStruct((seq, DT, btiles, 8, BLK), jnp.float32),
        mesh=mesh,
        scratch_types=[
            pltpu.VMEM((r, BLK), jnp.int32),
            pltpu.VMEM((BLK, EMB_DIM), jnp.float32),
            pltpu.VMEM((BLK, EMB_DIM), jnp.float32),
            pltpu.VMEM((DT, 8, BLK), jnp.float32),
            pltpu.VMEM((DT, 8, BLK), jnp.float32),
            pltpu.SemaphoreType.DMA,
            pltpu.SemaphoreType.DMA,
            pltpu.SemaphoreType.DMA,
            pltpu.SemaphoreType.DMA,
        ],
        compiler_params=pltpu.CompilerParams(
            use_tc_tiling_on_sc=False, needs_layout_passes=False),
    )
    def gather_kernel(idx_hbm, table_hbm, out_hbm, idx_v, gbuf0, gbuf1,
                      tbuf0, tbuf1, gs0, gs1, ws0, ws1):
        wid = lax.axis_index("s") * NC + lax.axis_index("c")
        blk0 = wid * r
        pltpu.sync_copy(idx_hbm.at[pl.ds(blk0, r)], idx_v)

        lane = lax.iota(jnp.int32, 16)

        def fire_gather(j, gbuf, gsem):
            pltpu.async_copy(table_hbm.at[idx_v.at[j]], gbuf, gsem)

        def wait_gather(gbuf, gsem):
            pltpu.make_async_copy(table_hbm.at[idx_v.at[0]], gbuf, gsem).wait()

        def transpose_block(gbuf, tbuf):
            def dt_body(dt, carry):
                for di in range(8):
                    col = jnp.full((16,), dt * 8 + di, jnp.int32)
                    for g in range(8):
                        vec = plsc.load_gather(gbuf, [lane + 16 * g, col])
                        tbuf[dt, di, pl.ds(16 * g, 16)] = vec
                return carry
            lax.fori_loop(0, DT, dt_body, 0)

        def out_slice(j):
            blk = blk0 + j
            s = blk // btiles
            bt = blk - s * btiles
            return out_hbm.at[s, :, bt]

        def fire_write(j, tbuf, wsem):
            pltpu.async_copy(tbuf, out_slice(j), wsem)

        def wait_write(tbuf, wsem):
            pltpu.make_async_copy(tbuf, out_slice(0), wsem).wait()

        fire_gather(0, gbuf0, gs0)

        def body(i, carry):
            j0 = 2 * i
            fire_gather(j0 + 1, gbuf1, gs1)
            wait_gather(gbuf0, gs0)

            @pl.when(i > 0)
            def _():
                wait_write(tbuf0, ws0)

            transpose_block(gbuf0, tbuf0)
            fire_write(j0, tbuf0, ws0)

            @pl.when(j0 + 2 < r)
            def _():
                fire_gather(j0 + 2, gbuf0, gs0)

            wait_gather(gbuf1, gs1)

            @pl.when(i > 0)
            def _():
                wait_write(tbuf1, ws1)

            transpose_block(gbuf1, tbuf1)
            fire_write(j0 + 1, tbuf1, ws1)
            return carry

        lax.fori_loop(0, r // 2, body, 0)
        wait_write(tbuf0, ws0)
        wait_write(tbuf1, ws1)

    return gather_kernel


def kernel(input, weight):
    b, s = input.shape
    assert b % BLK == 0
    btiles = b // BLK
    idx_blocks = input.astype(jnp.int32).T.reshape(s * btiles, BLK)
    out5 = _make_gather(s, btiles)(idx_blocks, weight)
    return out5.transpose(2, 4, 0, 1, 3).reshape(b, s, EMB_DIM)


# speedup vs baseline: 1.5431x; 1.5431x over previous
import functools

import jax
import jax.numpy as jnp
from jax import lax
from jax.experimental import pallas as pl
from jax.experimental.pallas import tpu as pltpu
from jax.experimental.pallas import tpu_sc as plsc

NC = 2
NS = 16
NW = NC * NS

EMB_DIM = 64
BLK = 128
DT = EMB_DIM // 8


def _make_gather(seq: int, btiles: int):
    nblk = seq * btiles
    assert nblk % (2 * NW) == 0
    r = nblk // NW

    mesh = plsc.VectorSubcoreMesh(core_axis_name="c", subcore_axis_name="s")

    @functools.partial(
        pl.kernel,
        out_type=jax.ShapeDtypeStruct((seq, DT, btiles, 8, BLK), jnp.float32),
        mesh=mesh,
        scratch_types=[
            pltpu.VMEM((r, BLK), jnp.int32),
            pltpu.VMEM((BLK, EMB_DIM), jnp.float32),
            pltpu.VMEM((BLK, EMB_DIM), jnp.float32),
            pltpu.VMEM((DT, 8, BLK), jnp.float32),
            pltpu.VMEM((DT, 8, BLK), jnp.float32),
            pltpu.SemaphoreType.DMA,
            pltpu.SemaphoreType.DMA,
            pltpu.SemaphoreType.DMA,
            pltpu.SemaphoreType.DMA,
        ],
        compiler_params=pltpu.CompilerParams(
            use_tc_tiling_on_sc=False, needs_layout_passes=False),
    )
    def gather_kernel(idx_hbm, table_hbm, out_hbm, idx_v, gbuf0, gbuf1,
                      tbuf0, tbuf1, gs0, gs1, ws0, ws1):
        wid = lax.axis_index("s") * NC + lax.axis_index("c")
        blk0 = wid * r
        pltpu.sync_copy(idx_hbm.at[pl.ds(blk0, r)], idx_v)

        lane = lax.iota(jnp.int32, 16)

        def fire_gather(j, gbuf, gsem):
            pltpu.async_copy(table_hbm.at[idx_v.at[j]], gbuf, gsem)

        def wait_gather(gbuf, gsem):
            pltpu.make_async_copy(table_hbm.at[idx_v.at[0]], gbuf, gsem).wait()

        def transpose_block(gbuf, tbuf):
            @plsc.parallel_loop(0, EMB_DIM, step=1, unroll=8)
            def _(d):
                dt = d // 8
                di = d - dt * 8
                col = jnp.full((16,), d, jnp.int32)
                for g in range(8):
                    vec = plsc.load_gather(gbuf, [lane + 16 * g, col])
                    tbuf[dt, di, pl.ds(16 * g, 16)] = vec

        def out_slice(j):
            blk = blk0 + j
            s = blk // btiles
            bt = blk - s * btiles
            return out_hbm.at[s, :, bt]

        def fire_write(j, tbuf, wsem):
            pltpu.async_copy(tbuf, out_slice(j), wsem)

        def wait_write(tbuf, wsem):
            pltpu.make_async_copy(tbuf, out_slice(0), wsem).wait()

        fire_gather(0, gbuf0, gs0)

        def body(i, carry):
            j0 = 2 * i
            fire_gather(j0 + 1, gbuf1, gs1)
            wait_gather(gbuf0, gs0)

            @pl.when(i > 0)
            def _():
                wait_write(tbuf0, ws0)

            transpose_block(gbuf0, tbuf0)
            fire_write(j0, tbuf0, ws0)

            @pl.when(j0 + 2 < r)
            def _():
                fire_gather(j0 + 2, gbuf0, gs0)

            wait_gather(gbuf1, gs1)

            @pl.when(i > 0)
            def _():
                wait_write(tbuf1, ws1)

            transpose_block(gbuf1, tbuf1)
            fire_write(j0 + 1, tbuf1, ws1)
            return carry

        lax.fori_loop(0, r // 2, body, 0)
        wait_write(tbuf0, ws0)
        wait_write(tbuf1, ws1)

    return gather_kernel


def kernel(input, weight):
    b, s = input.shape
    assert b % BLK == 0
    btiles = b // BLK
    idx_blocks = input.astype(jnp.int32).T.reshape(s * btiles, BLK)
    out5 = _make_gather(s, btiles)(idx_blocks, weight)
    return out5.transpose(2, 4, 0, 1, 3).reshape(b, s, EMB_DIM)


# speedup vs baseline: 2.8963x; 1.8769x over previous
import functools

import jax
import jax.numpy as jnp
from jax import lax
from jax.experimental import pallas as pl
from jax.experimental.pallas import tpu as pltpu
from jax.experimental.pallas import tpu_sc as plsc

NC = 2
NS = 16
NW = NC * NS

EMB_DIM = 64
BLK = 128
DT = EMB_DIM // 8


def _make_gather(seq: int, btiles: int):
    nblk = seq * btiles
    assert nblk % (2 * NW) == 0
    r = nblk // NW

    mesh = plsc.VectorSubcoreMesh(core_axis_name="c", subcore_axis_name="s")

    @functools.partial(
        pl.kernel,
        out_type=jax.ShapeDtypeStruct((seq, DT, btiles, 8, BLK), jnp.float32),
        mesh=mesh,
        scratch_types=[
            pltpu.VMEM((r, BLK), jnp.int32),
            pltpu.VMEM((BLK, EMB_DIM), jnp.float32),
            pltpu.VMEM((BLK, EMB_DIM), jnp.float32),
            pltpu.VMEM((DT, 8, BLK), jnp.float32),
            pltpu.VMEM((DT, 8, BLK), jnp.float32),
            pltpu.SemaphoreType.DMA,
            pltpu.SemaphoreType.DMA,
            pltpu.SemaphoreType.DMA,
            pltpu.SemaphoreType.DMA,
        ],
        compiler_params=pltpu.CompilerParams(
            use_tc_tiling_on_sc=False, needs_layout_passes=False),
    )
    def gather_kernel(idx_hbm, table_hbm, out_hbm, idx_v, gbuf0, gbuf1,
                      tbuf0, tbuf1, gs0, gs1, ws0, ws1):
        wid = lax.axis_index("s") * NC + lax.axis_index("c")
        blk0 = wid * r
        pltpu.sync_copy(idx_hbm.at[pl.ds(blk0, r)], idx_v)

        lane = lax.iota(jnp.int32, 16)

        def fire_gather(j, gbuf, gsem):
            pltpu.async_copy(table_hbm.at[idx_v.at[j]], gbuf, gsem)

        def wait_gather(gbuf, gsem):
            pltpu.make_async_copy(table_hbm.at[idx_v.at[0]], gbuf, gsem).wait()

        def transpose_block(gbuf, tbuf):
            @plsc.parallel_loop(0, 32, step=1, unroll=2)
            def _(tile):
                t0 = (tile % 8) * 16
                d0 = (tile // 8) * 16
                rows = t0 + lane
                for k in range(16):
                    cols = d0 + ((lane + k) % 16)
                    vec = plsc.load_gather(gbuf, [rows, cols])
                    plsc.store_scatter(tbuf, [cols // 8, cols % 8, rows],
                                       vec)

        def out_slice(j):
            blk = blk0 + j
            s = blk // btiles
            bt = blk - s * btiles
            return out_hbm.at[s, :, bt]

        def fire_write(j, tbuf, wsem):
            pltpu.async_copy(tbuf, out_slice(j), wsem)

        def wait_write(tbuf, wsem):
            pltpu.make_async_copy(tbuf, out_slice(0), wsem).wait()

        fire_gather(0, gbuf0, gs0)

        def body(i, carry):
            j0 = 2 * i
            fire_gather(j0 + 1, gbuf1, gs1)
            wait_gather(gbuf0, gs0)

            @pl.when(i > 0)
            def _():
                wait_write(tbuf0, ws0)

            transpose_block(gbuf0, tbuf0)
            fire_write(j0, tbuf0, ws0)

            @pl.when(j0 + 2 < r)
            def _():
                fire_gather(j0 + 2, gbuf0, gs0)

            wait_gather(gbuf1, gs1)

            @pl.when(i > 0)
            def _():
                wait_write(tbuf1, ws1)

            transpose_block(gbuf1, tbuf1)
            fire_write(j0 + 1, tbuf1, ws1)
            return carry

        lax.fori_loop(0, r // 2, body, 0)
        wait_write(tbuf0, ws0)
        wait_write(tbuf1, ws1)

    return gather_kernel


def kernel(input, weight):
    b, s = input.shape
    assert b % BLK == 0
    btiles = b // BLK
    idx_blocks = input.astype(jnp.int32).T.reshape(s * btiles, BLK)
    out5 = _make_gather(s, btiles)(idx_blocks, weight)
    return out5.transpose(2, 4, 0, 1, 3).reshape(b, s, EMB_DIM)


# speedup vs baseline: 2.9184x; 1.0076x over previous
import functools

import jax
import jax.numpy as jnp
from jax import lax
from jax.experimental import pallas as pl
from jax.experimental.pallas import tpu as pltpu
from jax.experimental.pallas import tpu_sc as plsc

NC = 2
NS = 16
NW = NC * NS

EMB_DIM = 64
BLK = 128
DT = EMB_DIM // 8
NSLOT = 5


def _make_gather(seq: int, btiles: int):
    nblk = seq * btiles
    assert nblk % (NW * NSLOT) == 0
    r = nblk // NW

    mesh = plsc.VectorSubcoreMesh(core_axis_name="c", subcore_axis_name="s")

    @functools.partial(
        pl.kernel,
        out_type=jax.ShapeDtypeStruct((seq, DT, btiles, 8, BLK), jnp.float32),
        mesh=mesh,
        scratch_types=[
            pltpu.VMEM((r, BLK), jnp.int32),
            pltpu.VMEM((NSLOT, BLK, EMB_DIM), jnp.float32),
            pltpu.VMEM((NSLOT, DT, 8, BLK), jnp.float32),
            pltpu.SemaphoreType.DMA((NSLOT,)),
            pltpu.SemaphoreType.DMA((NSLOT,)),
        ],
        compiler_params=pltpu.CompilerParams(
            use_tc_tiling_on_sc=False, needs_layout_passes=False),
    )
    def gather_kernel(idx_hbm, table_hbm, out_hbm, idx_v, gbuf, tbuf, gs, ws):
        wid = lax.axis_index("s") * NC + lax.axis_index("c")
        blk0 = wid * r
        pltpu.sync_copy(idx_hbm.at[pl.ds(blk0, r)], idx_v)

        lane = lax.iota(jnp.int32, 16)

        def fire_gather(j, q):
            pltpu.async_copy(table_hbm.at[idx_v.at[j]], gbuf.at[q], gs.at[q])

        def wait_gather(q):
            pltpu.make_async_copy(table_hbm.at[idx_v.at[0]], gbuf.at[q],
                                  gs.at[q]).wait()

        def transpose_block(q):
            g = gbuf.at[q]
            t = tbuf.at[q]

            @plsc.parallel_loop(0, 32, step=1, unroll=2)
            def _(tile):
                t0 = (tile % 8) * 16
                d0 = (tile // 8) * 16
                rows = t0 + lane
                for k in range(16):
                    cols = d0 + ((lane + k) % 16)
                    vec = plsc.load_gather(g, [rows, cols])
                    plsc.store_scatter(t, [cols // 8, cols % 8, rows], vec)

        def out_slice(j):
            blk = blk0 + j
            s = blk // btiles
            bt = blk - s * btiles
            return out_hbm.at[s, :, bt]

        def fire_write(j, q):
            pltpu.async_copy(tbuf.at[q], out_slice(j), ws.at[q])

        def wait_write(q):
            pltpu.make_async_copy(tbuf.at[q], out_slice(0), ws.at[q]).wait()

        for q in range(NSLOT):
            fire_gather(q, q)

        def body(i, carry):
            for q in range(NSLOT):
                j = NSLOT * i + q
                wait_gather(q)

                @pl.when(i > 0)
                def _():
                    wait_write(q)

                transpose_block(q)
                fire_write(j, q)

                @pl.when(i < (r // NSLOT) - 1)
                def _():
                    fire_gather(j + NSLOT, q)
            return carry

        lax.fori_loop(0, r // NSLOT, body, 0)
        for q in range(NSLOT):
            wait_write(q)

    return gather_kernel


def kernel(input, weight):
    b, s = input.shape
    assert b % BLK == 0
    btiles = b // BLK
    idx_blocks = input.astype(jnp.int32).T.reshape(s * btiles, BLK)
    out5 = _make_gather(s, btiles)(idx_blocks, weight)
    return out5.transpose(2, 4, 0, 1, 3).reshape(b, s, EMB_DIM)


# speedup vs baseline: 2.9482x; 1.0102x over previous
import functools

import jax
import jax.numpy as jnp
from jax import lax
from jax.experimental import pallas as pl
from jax.experimental.pallas import tpu as pltpu
from jax.experimental.pallas import tpu_sc as plsc

NC = 2
NS = 16
NW = NC * NS

EMB_DIM = 64
BLK = 128
DT = EMB_DIM // 8
NSLOT = 5


def _make_gather(seq: int, btiles: int):
    nblk = seq * btiles
    assert nblk % (NW * NSLOT) == 0
    r = nblk // NW

    mesh = plsc.VectorSubcoreMesh(core_axis_name="c", subcore_axis_name="s")

    @functools.partial(
        pl.kernel,
        out_type=jax.ShapeDtypeStruct((seq, DT, btiles, 8, BLK), jnp.float32),
        mesh=mesh,
        scratch_types=[
            pltpu.VMEM((r, BLK), jnp.int32),
            pltpu.VMEM((NSLOT, BLK, 2 * EMB_DIM), jnp.float32),
            pltpu.VMEM((NSLOT, DT, 8, BLK), jnp.float32),
            pltpu.SemaphoreType.DMA((NSLOT,)),
            pltpu.SemaphoreType.DMA((NSLOT,)),
        ],
        compiler_params=pltpu.CompilerParams(
            use_tc_tiling_on_sc=False, needs_layout_passes=False),
    )
    def gather_kernel(idx_hbm, table_hbm, out_hbm, idx_v, gbuf, tbuf, gs, ws):
        wid = lax.axis_index("s") * NC + lax.axis_index("c")
        blk0 = wid * r
        pltpu.sync_copy(idx_hbm.at[pl.ds(blk0, r)], idx_v)

        lane = lax.iota(jnp.int32, 16)

        def fire_gather(j, q):
            pltpu.async_copy(table_hbm.at[idx_v.at[j]], gbuf.at[q], gs.at[q])

        def wait_gather(q):
            pltpu.make_async_copy(table_hbm.at[idx_v.at[0]], gbuf.at[q],
                                  gs.at[q]).wait()

        def transpose_block(q):
            g = gbuf.at[q]
            t = tbuf.at[q]

            @plsc.parallel_loop(0, 32, step=1, unroll=2)
            def _(tile):
                t0 = (tile % 8) * 16
                d0 = (tile // 8) * 16
                rows = t0 + lane
                for k in range(16):
                    cols = d0 + ((lane + k) % 16)
                    vec = plsc.load_gather(g, [rows, cols])
                    plsc.store_scatter(t, [cols // 8, cols % 8, rows], vec)

        def out_slice(j):
            blk = blk0 + j
            s = blk // btiles
            bt = blk - s * btiles
            return out_hbm.at[s, :, bt]

        def fire_write(j, q):
            pltpu.async_copy(tbuf.at[q], out_slice(j), ws.at[q])

        def wait_write(q):
            pltpu.make_async_copy(tbuf.at[q], out_slice(0), ws.at[q]).wait()

        for q in range(NSLOT):
            fire_gather(q, q)

        def body(i, carry):
            for q in range(NSLOT):
                j = NSLOT * i + q
                wait_gather(q)

                @pl.when(i > 0)
                def _():
                    wait_write(q)

                transpose_block(q)
                fire_write(j, q)

                @pl.when(i < (r // NSLOT) - 1)
                def _():
                    fire_gather(j + NSLOT, q)
            return carry

        lax.fori_loop(0, r // NSLOT, body, 0)
        for q in range(NSLOT):
            wait_write(q)

    return gather_kernel


def kernel(input, weight):
    b, s = input.shape
    assert b % BLK == 0
    btiles = b // BLK
    idx_blocks = input.astype(jnp.int32).T.reshape(s * btiles, BLK)
    wpad = jnp.pad(weight, ((0, 0), (0, EMB_DIM)))
    out5 = _make_gather(s, btiles)(idx_blocks, wpad)
    return out5.transpose(2, 4, 0, 1, 3).reshape(b, s, EMB_DIM)


# speedup vs baseline: 3.6317x; 1.2318x over previous
import functools

import jax
import jax.numpy as jnp
from jax import lax
from jax.experimental import pallas as pl
from jax.experimental.pallas import tpu as pltpu
from jax.experimental.pallas import tpu_sc as plsc

NC = 2
NS = 16
NW = NC * NS

EMB_DIM = 64
BLK = 128
DT = EMB_DIM // 8
NSLOT = 5


def _make_tablefix(vocab: int):
    full = vocab // BLK
    vtail = vocab - full * BLK
    iters = (full + NW - 1) // NW

    mesh = plsc.VectorSubcoreMesh(core_axis_name="c", subcore_axis_name="s")

    @functools.partial(
        pl.kernel,
        out_type=jax.ShapeDtypeStruct((vocab // 2, BLK), jnp.float32),
        mesh=mesh,
        scratch_types=[
            pltpu.VMEM((2, EMB_DIM, BLK), jnp.float32),
            pltpu.VMEM((2, EMB_DIM, BLK), jnp.float32),
            pltpu.SemaphoreType.DMA((2,)),
            pltpu.SemaphoreType.DMA((2,)),
            pltpu.SemaphoreType.DMA,
        ],
        compiler_params=pltpu.CompilerParams(
            use_tc_tiling_on_sc=True, needs_layout_passes=False),
    )
    def tablefix_kernel(wt_hbm, tail_hbm, out_hbm, ibuf, obuf, rs, ws, ts):
        wid = lax.axis_index("s") * NC + lax.axis_index("c")
        lane = lax.iota(jnp.int32, 16)

        def fire_read(c, q):
            pltpu.async_copy(wt_hbm.at[:, pl.ds(c * BLK, BLK)], ibuf.at[q],
                             rs.at[q])

        def wait_read(q):
            pltpu.make_async_copy(wt_hbm.at[:, pl.ds(0, BLK)], ibuf.at[q],
                                  rs.at[q]).wait()

        def transpose_chunk(q):
            g = ibuf.at[q]
            t = obuf.at[q]

            @plsc.parallel_loop(0, 32, step=1, unroll=2)
            def _(it):
                v00 = (it % 8) * 16
                d0 = (it // 8) * 16
                voff = v00 + lane
                for k in range(16):
                    d = d0 + ((lane + k) % 16)
                    vec = plsc.load_gather(g, [d, voff])
                    plsc.store_scatter(
                        t, [voff // 2, (voff % 2) * EMB_DIM + d], vec)

        def fire_write(c, q):
            pltpu.async_copy(obuf.at[q],
                             out_hbm.at[pl.ds(c * (BLK // 2), BLK // 2)],
                             ws.at[q])

        def wait_write(q):
            pltpu.make_async_copy(obuf.at[q],
                                  out_hbm.at[pl.ds(0, BLK // 2)],
                                  ws.at[q]).wait()

        @pl.when(wid < full)
        def _():
            fire_read(wid, 0)

        @pl.when(wid + NW < full)
        def _():
            fire_read(wid + NW, 1)

        def body(i, carry):
            for q in range(2):
                c = wid + (2 * i + q) * NW

                @pl.when(c < full)
                def _():
                    wait_read(q)

                    @pl.when(i > 0)
                    def _():
                        wait_write(q)

                    transpose_chunk(q)
                    fire_write(c, q)

                @pl.when(c + 2 * NW < full)
                def _():
                    fire_read(c + 2 * NW, q)
            return carry

        lax.fori_loop(0, (iters + 1) // 2, body, 0)
        for q in range(2):
            @pl.when(wid + q * NW < full)
            def _():
                wait_write(q)

        @pl.when((wid == 0) & (vtail > 0))
        def _():
            pltpu.async_copy(
                tail_hbm,
                out_hbm.at[pl.ds(full * (BLK // 2), vtail // 2)], ts).wait()

    return tablefix_kernel


def _make_gather(seq: int, btiles: int):
    nblk = seq * btiles
    assert nblk % (NW * NSLOT) == 0
    r = nblk // NW

    mesh = plsc.VectorSubcoreMesh(core_axis_name="c", subcore_axis_name="s")

    @functools.partial(
        pl.kernel,
        out_type=jax.ShapeDtypeStruct((seq, DT, btiles, 8, BLK), jnp.float32),
        mesh=mesh,
        scratch_types=[
            pltpu.VMEM((r, BLK), jnp.int32),
            pltpu.VMEM((NSLOT, BLK, EMB_DIM), jnp.float32),
            pltpu.VMEM((NSLOT, DT, 8, BLK), jnp.float32),
            pltpu.SemaphoreType.DMA((NSLOT,)),
            pltpu.SemaphoreType.DMA((NSLOT,)),
        ],
        compiler_params=pltpu.CompilerParams(
            use_tc_tiling_on_sc=False, needs_layout_passes=False),
    )
    def gather_kernel(idx_hbm, table_hbm, out_hbm, idx_v, gbuf, tbuf, gs, ws):
        wid = lax.axis_index("s") * NC + lax.axis_index("c")
        blk0 = wid * r
        pltpu.sync_copy(idx_hbm.at[pl.ds(blk0, r)], idx_v)

        lane = lax.iota(jnp.int32, 16)

        def fire_gather(j, q):
            pltpu.async_copy(table_hbm.at[idx_v.at[j]], gbuf.at[q], gs.at[q])

        def wait_gather(q):
            pltpu.make_async_copy(table_hbm.at[idx_v.at[0]], gbuf.at[q],
                                  gs.at[q]).wait()

        def transpose_block(q):
            g = gbuf.at[q]
            t = tbuf.at[q]

            @plsc.parallel_loop(0, 32, step=1, unroll=2)
            def _(tile):
                t0 = (tile % 8) * 16
                d0 = (tile // 8) * 16
                rows = t0 + lane
                for k in range(16):
                    cols = d0 + ((lane + k) % 16)
                    vec = plsc.load_gather(g, [rows, cols])
                    plsc.store_scatter(t, [cols // 8, cols % 8, rows], vec)

        def out_slice(j):
            blk = blk0 + j
            s = blk // btiles
            bt = blk - s * btiles
            return out_hbm.at[s, :, bt]

        def fire_write(j, q):
            pltpu.async_copy(tbuf.at[q], out_slice(j), ws.at[q])

        def wait_write(q):
            pltpu.make_async_copy(tbuf.at[q], out_slice(0), ws.at[q]).wait()

        for q in range(NSLOT):
            fire_gather(q, q)

        def body(i, carry):
            for q in range(NSLOT):
                j = NSLOT * i + q
                wait_gather(q)

                @pl.when(i > 0)
                def _():
                    wait_write(q)

                transpose_block(q)
                fire_write(j, q)

                @pl.when(i < (r // NSLOT) - 1)
                def _():
                    fire_gather(j + NSLOT, q)
            return carry

        lax.fori_loop(0, r // NSLOT, body, 0)
        for q in range(NSLOT):
            wait_write(q)

    return gather_kernel


def kernel(input, weight):
    b, s = input.shape
    assert b % BLK == 0
    btiles = b // BLK
    vocab = weight.shape[0]
    full = vocab // BLK
    tail = weight[full * BLK:].reshape(-1, 2 * EMB_DIM)
    wc = _make_tablefix(vocab)(weight.T, tail)
    table = wc.reshape(vocab, EMB_DIM)
    idx_blocks = input.astype(jnp.int32).T.reshape(s * btiles, BLK)
    out5 = _make_gather(s, btiles)(idx_blocks, table)
    return out5.transpose(2, 4, 0, 1, 3).reshape(b, s, EMB_DIM)


# speedup vs baseline: 3.7356x; 1.0286x over previous
import functools

import jax
import jax.numpy as jnp
from jax import lax
from jax.experimental import pallas as pl
from jax.experimental.pallas import tpu as pltpu
from jax.experimental.pallas import tpu_sc as plsc

NC = 2
NS = 16
NW = NC * NS

EMB_DIM = 64
BLK = 128
DT = EMB_DIM // 8
NSLOT = 5


def _make_tablefix(vocab: int):
    full = vocab // BLK
    vtail = vocab - full * BLK
    iters = (full + NW - 1) // NW

    mesh = plsc.VectorSubcoreMesh(core_axis_name="c", subcore_axis_name="s")

    @functools.partial(
        pl.kernel,
        out_type=jax.ShapeDtypeStruct((vocab // 2, BLK), jnp.float32),
        mesh=mesh,
        scratch_types=[
            pltpu.VMEM((NSLOT, EMB_DIM, BLK), jnp.float32),
            pltpu.VMEM((NSLOT, EMB_DIM, BLK), jnp.float32),
            pltpu.SemaphoreType.DMA((NSLOT,)),
            pltpu.SemaphoreType.DMA((NSLOT,)),
            pltpu.SemaphoreType.DMA,
        ],
        compiler_params=pltpu.CompilerParams(
            use_tc_tiling_on_sc=True, needs_layout_passes=False),
    )
    def tablefix_kernel(wt_hbm, tail_hbm, out_hbm, ibuf, obuf, rs, ws, ts):
        wid = lax.axis_index("s") * NC + lax.axis_index("c")
        lane = lax.iota(jnp.int32, 16)

        def fire_read(c, q):
            pltpu.async_copy(wt_hbm.at[:, pl.ds(c * BLK, BLK)], ibuf.at[q],
                             rs.at[q])

        def wait_read(q):
            pltpu.make_async_copy(wt_hbm.at[:, pl.ds(0, BLK)], ibuf.at[q],
                                  rs.at[q]).wait()

        def transpose_chunk(q):
            g = ibuf.at[q]
            t = obuf.at[q]

            @plsc.parallel_loop(0, 32, step=1, unroll=2)
            def _(it):
                v00 = (it % 8) * 16
                d0 = (it // 8) * 16
                voff = v00 + lane
                for k in range(16):
                    d = d0 + ((lane + k) % 16)
                    vec = plsc.load_gather(g, [d, voff])
                    plsc.store_scatter(
                        t, [voff // 2, (voff % 2) * EMB_DIM + d], vec)

        def fire_write(c, q):
            pltpu.async_copy(obuf.at[q],
                             out_hbm.at[pl.ds(c * (BLK // 2), BLK // 2)],
                             ws.at[q])

        def wait_write(q):
            pltpu.make_async_copy(obuf.at[q],
                                  out_hbm.at[pl.ds(0, BLK // 2)],
                                  ws.at[q]).wait()

        for q in range(NSLOT):
            @pl.when(wid + q * NW < full)
            def _():
                fire_read(wid + q * NW, q)

        groups = (iters + NSLOT - 1) // NSLOT

        def body(i, carry):
            for q in range(NSLOT):
                c = wid + (NSLOT * i + q) * NW

                @pl.when(c < full)
                def _():
                    wait_read(q)

                    @pl.when(i > 0)
                    def _():
                        wait_write(q)

                    transpose_chunk(q)
                    fire_write(c, q)

                @pl.when(c + NSLOT * NW < full)
                def _():
                    fire_read(c + NSLOT * NW, q)
            return carry

        lax.fori_loop(0, groups, body, 0)
        for q in range(NSLOT):
            @pl.when(wid + q * NW < full)
            def _():
                wait_write(q)

        @pl.when((wid == 0) & (vtail > 0))
        def _():
            pltpu.async_copy(
                tail_hbm,
                out_hbm.at[pl.ds(full * (BLK // 2), vtail // 2)], ts).wait()

    return tablefix_kernel


def _make_gather(seq: int, btiles: int):
    nblk = seq * btiles
    assert nblk % (NW * NSLOT) == 0
    r = nblk // NW

    mesh = plsc.VectorSubcoreMesh(core_axis_name="c", subcore_axis_name="s")

    @functools.partial(
        pl.kernel,
        out_type=jax.ShapeDtypeStruct((seq, DT, btiles, 8, BLK), jnp.float32),
        mesh=mesh,
        scratch_types=[
            pltpu.VMEM((r, BLK), jnp.int32),
            pltpu.VMEM((NSLOT, BLK, EMB_DIM), jnp.float32),
            pltpu.VMEM((NSLOT, DT, 8, BLK), jnp.float32),
            pltpu.SemaphoreType.DMA((NSLOT,)),
            pltpu.SemaphoreType.DMA((NSLOT,)),
        ],
        compiler_params=pltpu.CompilerParams(
            use_tc_tiling_on_sc=False, needs_layout_passes=False),
    )
    def gather_kernel(idx_hbm, table_hbm, out_hbm, idx_v, gbuf, tbuf, gs, ws):
        wid = lax.axis_index("s") * NC + lax.axis_index("c")
        blk0 = wid * r
        pltpu.sync_copy(idx_hbm.at[pl.ds(blk0, r)], idx_v)

        lane = lax.iota(jnp.int32, 16)

        def fire_gather(j, q):
            pltpu.async_copy(table_hbm.at[idx_v.at[j]], gbuf.at[q], gs.at[q])

        def wait_gather(q):
            pltpu.make_async_copy(table_hbm.at[idx_v.at[0]], gbuf.at[q],
                                  gs.at[q]).wait()

        def transpose_block(q):
            g = gbuf.at[q]
            t = tbuf.at[q]

            @plsc.parallel_loop(0, 32, step=1, unroll=2)
            def _(tile):
                t0 = (tile % 8) * 16
                d0 = (tile // 8) * 16
                rows = t0 + lane
                for k in range(16):
                    cols = d0 + ((lane + k) % 16)
                    vec = plsc.load_gather(g, [rows, cols])
                    plsc.store_scatter(t, [cols // 8, cols % 8, rows], vec)

        def out_slice(j):
            blk = blk0 + j
            s = blk // btiles
            bt = blk - s * btiles
            return out_hbm.at[s, :, bt]

        def fire_write(j, q):
            pltpu.async_copy(tbuf.at[q], out_slice(j), ws.at[q])

        def wait_write(q):
            pltpu.make_async_copy(tbuf.at[q], out_slice(0), ws.at[q]).wait()

        for q in range(NSLOT):
            fire_gather(q, q)

        def body(i, carry):
            for q in range(NSLOT):
                j = NSLOT * i + q
                wait_gather(q)

                @pl.when(i > 0)
                def _():
                    wait_write(q)

                transpose_block(q)
                fire_write(j, q)

                @pl.when(i < (r // NSLOT) - 1)
                def _():
                    fire_gather(j + NSLOT, q)
            return carry

        lax.fori_loop(0, r // NSLOT, body, 0)
        for q in range(NSLOT):
            wait_write(q)

    return gather_kernel


def kernel(input, weight):
    b, s = input.shape
    assert b % BLK == 0
    btiles = b // BLK
    vocab = weight.shape[0]
    full = vocab // BLK
    tail = weight[full * BLK:].reshape(-1, 2 * EMB_DIM)
    wc = _make_tablefix(vocab)(weight.T, tail)
    table = wc.reshape(vocab, EMB_DIM)
    idx_blocks = input.astype(jnp.int32).T.reshape(s * btiles, BLK)
    out5 = _make_gather(s, btiles)(idx_blocks, table)
    return out5.transpose(2, 4, 0, 1, 3).reshape(b, s, EMB_DIM)
